# Initial kernel scaffold; baseline (speedup 1.0000x reference)
import jax
import jax.numpy as jnp
from jax.experimental import pallas as pl


def kernel(x, edge_index, edge_weight, W_l, b_l, W_r, b_r, att, b_gat, W_xz, b_xz, W_hz, b_hz, W_xr, b_xr, W_hr, b_hr, W_xh, b_xh, W_hh, b_hh, W_out, b_out):
    raise NotImplementedError("write your pallas kernel here")


# speedup vs baseline: 3.6063x; 3.6063x over previous
import jax
import jax.numpy as jnp
from jax.experimental import pallas as pl
from jax.experimental.pallas import tpu as pltpu


def _gru_block_kernel(x_ref, Wx_ref, Whzr_ref, Whh_ref, bx_ref, Wout_ref,
                      bout_ref, out_ref, h_ref):
    win, B, F = x_ref.shape
    H = Whh_ref.shape[0]
    xall = x_ref[...].reshape(win * B, F)
    xproj = (jnp.dot(xall, Wx_ref[...], preferred_element_type=jnp.float32)
             + bx_ref[...])
    xproj = xproj.reshape(win, B, 3 * H)
    h = jnp.zeros((B, H), jnp.float32)
    for t in range(win):
        xp = xproj[t]
        zr = jnp.dot(h, Whzr_ref[...], preferred_element_type=jnp.float32)
        z = jax.nn.sigmoid(xp[:, :H] + zr[:, :H])
        r = jax.nn.sigmoid(xp[:, H:2 * H] + zr[:, H:])
        hc = jnp.dot(h * r, Whh_ref[...], preferred_element_type=jnp.float32)
        h_tilde = jnp.tanh(xp[:, 2 * H:] + hc)
        h = z * h + (1.0 - z) * h_tilde
    h_ref[...] = h
    out_ref[...] = (jnp.dot(h, Wout_ref[...], preferred_element_type=jnp.float32)
                    + bout_ref[...])


def kernel(x, edge_index, edge_weight, W_l, b_l, W_r, b_r, att, b_gat,
           W_xz, b_xz, W_hz, b_hz, W_xr, b_xr, W_hr, b_hr, W_xh, b_xh,
           W_hh, b_hh, W_out, b_out):
    win, n, f = x.shape
    hid = W_hz.shape[0]
    block = 1000
    grid = n // block

    Wx = jnp.concatenate([W_xz, W_xr, W_xh], axis=1)
    Whzr = jnp.concatenate([W_hz, W_hr], axis=1)
    bx = jnp.concatenate([b_xz + b_hz, b_xr + b_hr, b_xh + b_hh])[None, :]
    bout = b_out[None, :]

    out2d, h = pl.pallas_call(
        _gru_block_kernel,
        grid=(grid,),
        in_specs=[
            pl.BlockSpec((win, block, f), lambda i: (0, i, 0)),
            pl.BlockSpec((f, 3 * hid), lambda i: (0, 0)),
            pl.BlockSpec((hid, 2 * hid), lambda i: (0, 0)),
            pl.BlockSpec((hid, hid), lambda i: (0, 0)),
            pl.BlockSpec((1, 3 * hid), lambda i: (0, 0)),
            pl.BlockSpec((hid, 1), lambda i: (0, 0)),
            pl.BlockSpec((1, 1), lambda i: (0, 0)),
        ],
        out_specs=[
            pl.BlockSpec((block, 1), lambda i: (i, 0)),
            pl.BlockSpec((block, hid), lambda i: (i, 0)),
        ],
        out_shape=[
            jax.ShapeDtypeStruct((n, 1), jnp.float32),
            jax.ShapeDtypeStruct((n, hid), jnp.float32),
        ],
        compiler_params=pltpu.CompilerParams(
            dimension_semantics=("parallel",),
        ),
    )(x, Wx, Whzr, W_hh, bx, W_out, bout)
    return out2d[:, 0], h
